# baseline (speedup 1.0000x reference)
import functools

import jax
import jax.numpy as jnp
from jax import lax
from jax.experimental import pallas as pl
from jax.experimental.pallas import tpu as pltpu
from jax.experimental.pallas import tpu_sc as plsc

N_ANCHOR = 20000
N_GT = 64
BATCH = 4
L = 16
N_CHUNKS = 8
CHUNK = 2512
N_VEC = CHUNK // L
GTB = 4
N_GTB = N_GT // GTB
POS_IOU = 0.7
NEG_IOU = 0.3
BIG_I = 2**30

_f32 = jnp.float32
_i32 = jnp.int32
_LN2 = 0.6931471805599453
_SQRT2 = 1.4142135623730951


def _vlog(x):
    bits = plsc.bitcast(x, _i32)
    e = jnp.right_shift(bits, 23) - 127
    m = plsc.bitcast(jnp.bitwise_or(jnp.bitwise_and(bits, 0x7FFFFF),
                                    0x3F800000), _f32)
    big = m > _SQRT2
    m = jnp.where(big, m * 0.5, m)
    e = jnp.where(big, e + 1, e)
    z = (m - 1.0) / (m + 1.0)
    z2 = z * z
    p = ((z2 * (1.0 / 7.0) + (1.0 / 5.0)) * z2 + (1.0 / 3.0)) * z2 + 1.0
    return e.astype(_f32) * _LN2 + 2.0 * z * p


def _sl1(d):
    return jnp.where(d < 1.0, 0.5 * d * d, d - 0.5)


def _sc_body(anc, bbox, loc, score, out,
             ax1, ay1, ax2, ay2, l0, l1, l2, l3, sc0, sc1,
             bx1, by1, bx2, by2,
             miou, rga, rls, ce0s, ce1s, flags, colv, coli, scat,
             mv, mi, ms, gt_arg, stage, fin, outv,
             sh_colv, sh_coli, sh_scat, sh_sums):
    c = lax.axis_index("c")
    s = lax.axis_index("s")
    img_l = jnp.right_shift(s, 3)
    img = 2 * c + img_l
    chunk = jnp.bitwise_and(s, 7)
    base = chunk * CHUNK
    lanes = lax.broadcasted_iota(_i32, (L,), 0)

    NP = N_CHUNKS * CHUNK
    pltpu.sync_copy(anc.at[pl.ds(0 * NP + base, CHUNK)], ax1)
    pltpu.sync_copy(anc.at[pl.ds(1 * NP + base, CHUNK)], ay1)
    pltpu.sync_copy(anc.at[pl.ds(2 * NP + base, CHUNK)], ax2)
    pltpu.sync_copy(anc.at[pl.ds(3 * NP + base, CHUNK)], ay2)
    pltpu.sync_copy(loc.at[pl.ds((img * 4 + 0) * NP + base, CHUNK)], l0)
    pltpu.sync_copy(loc.at[pl.ds((img * 4 + 1) * NP + base, CHUNK)], l1)
    pltpu.sync_copy(loc.at[pl.ds((img * 4 + 2) * NP + base, CHUNK)], l2)
    pltpu.sync_copy(loc.at[pl.ds((img * 4 + 3) * NP + base, CHUNK)], l3)
    pltpu.sync_copy(score.at[pl.ds((img * 2 + 0) * NP + base, CHUNK)], sc0)
    pltpu.sync_copy(score.at[pl.ds((img * 2 + 1) * NP + base, CHUNK)], sc1)
    pltpu.sync_copy(bbox.at[pl.ds((img * 4 + 0) * N_GT, N_GT)], bx1)
    pltpu.sync_copy(bbox.at[pl.ds((img * 4 + 1) * N_GT, N_GT)], by1)
    pltpu.sync_copy(bbox.at[pl.ds((img * 4 + 2) * N_GT, N_GT)], bx2)
    pltpu.sync_copy(bbox.at[pl.ds((img * 4 + 3) * N_GT, N_GT)], by2)

    def _init_scat(j, _):
        scat[pl.ds(j * L, L)] = jnp.full((L,), -1, _i32)
        return 0

    lax.fori_loop(0, N_GT, _init_scat, 0)

    def _init_flags(i, _):
        flags[pl.ds(i * L, L)] = jnp.zeros((L,), _i32)
        return 0

    lax.fori_loop(0, N_VEC, _init_flags, 0)

    for gtb in range(N_GTB):
        blk = (gtb * GTB) // L
        off16 = blk * L
        sub = (gtb * GTB) % L
        b1v = bx1[pl.ds(off16, L)]
        b2v = by1[pl.ds(off16, L)]
        b3v = bx2[pl.ds(off16, L)]
        b4v = by2[pl.ds(off16, L)]
        abv = (b3v - b1v) * (b4v - b2v)
        zsplat = jnp.zeros((L,), _f32)
        gb1 = [zsplat + b1v[sub + j] for j in range(GTB)]
        gb2 = [zsplat + b2v[sub + j] for j in range(GTB)]
        gb3 = [zsplat + b3v[sub + j] for j in range(GTB)]
        gb4 = [zsplat + b4v[sub + j] for j in range(GTB)]
        gab = [zsplat + abv[sub + j] for j in range(GTB)]

        def _main(i, col, gtb=gtb, gb1=gb1, gb2=gb2, gb3=gb3, gb4=gb4,
                  gab=gab):
            off = i * L
            a1 = ax1[pl.ds(off, L)]
            a2 = ay1[pl.ds(off, L)]
            a3 = ax2[pl.ds(off, L)]
            a4 = ay2[pl.ds(off, L)]
            aidx = base + off + lanes
            area_a = (a3 - a1) * (a4 - a2)
            if gtb == 0:
                rmax = jnp.full((L,), -1.0, _f32)
                rg = jnp.zeros((L,), _i32)
            else:
                rmax = miou[pl.ds(off, L)]
                rg = rga[pl.ds(off, L)]
            cvs = list(col)
            for j in range(GTB):
                g = gtb * GTB + j
                iw = jnp.maximum(
                    jnp.minimum(a3, gb3[j]) - jnp.maximum(a1, gb1[j]), 0.0)
                ih = jnp.maximum(
                    jnp.minimum(a4, gb4[j]) - jnp.maximum(a2, gb2[j]), 0.0)
                inter = iw * ih
                iou = inter / (area_a + gab[j] - inter + 1e-9)
                better = iou > rmax
                rmax = jnp.where(better, iou, rmax)
                rg = jnp.where(better, g, rg)
                cb = iou > cvs[2 * j]
                cvs[2 * j] = jnp.where(cb, iou, cvs[2 * j])
                cvs[2 * j + 1] = jnp.where(cb, aidx, cvs[2 * j + 1])
            miou[pl.ds(off, L)] = rmax
            rga[pl.ds(off, L)] = rg
            return tuple(cvs)

        col0 = []
        for j in range(GTB):
            col0.append(jnp.full((L,), -1.0, _f32))
            col0.append(jnp.zeros((L,), _i32))
        colf = lax.fori_loop(0, N_VEC, _main, tuple(col0))
        for j in range(GTB):
            g = gtb * GTB + j
            colv[pl.ds(g * L, L)] = colf[2 * j]
            coli[pl.ds(g * L, L)] = colf[2 * j + 1]

    def _tail(i, _):
        off = i * L
        aidx = base + off + lanes
        valid = aidx < N_ANCHOR
        rg = rga[pl.ds(off, L)]
        plsc.store_scatter(scat, [rg * L + lanes], aidx, mask=valid)
        a1 = ax1[pl.ds(off, L)]
        a2 = ay1[pl.ds(off, L)]
        a3 = ax2[pl.ds(off, L)]
        a4 = ay2[pl.ds(off, L)]
        m1 = plsc.load_gather(bx1, [rg])
        m2 = plsc.load_gather(by1, [rg])
        m3 = plsc.load_gather(bx2, [rg])
        m4 = plsc.load_gather(by2, [rg])
        eps = jnp.finfo(_f32).eps
        w = a3 - a1
        h = a4 - a2
        cx = a1 + w * 0.5
        cy = a2 + h * 0.5
        dw_ = m3 - m1
        dh_ = m4 - m2
        dcx = m1 + dw_ * 0.5
        dcy = m2 + dh_ * 0.5
        w = jnp.maximum(w, eps)
        h = jnp.maximum(h, eps)
        tdx = (dcx - cx) / w
        tdy = (dcy - cy) / h
        tdw = _vlog(dw_ / w)
        tdh = _vlog(dh_ / h)
        rl = (_sl1(jnp.abs(tdx - l0[pl.ds(off, L)]))
              + _sl1(jnp.abs(tdy - l1[pl.ds(off, L)]))
              + _sl1(jnp.abs(tdw - l2[pl.ds(off, L)]))
              + _sl1(jnp.abs(tdh - l3[pl.ds(off, L)])))
        s0 = sc0[pl.ds(off, L)]
        s1 = sc1[pl.ds(off, L)]
        mx = jnp.maximum(s0, s1)
        lse = mx + _vlog(1.0 + jnp.exp(-jnp.abs(s0 - s1)))
        rls[pl.ds(off, L)] = rl
        ce0s[pl.ds(off, L)] = lse - s0
        ce1s[pl.ds(off, L)] = lse - s1
        return 0

    lax.fori_loop(0, N_VEC, _tail, 0)

    TBL = N_GT * L
    tb = img_l * N_CHUNKS * TBL
    pltpu.sync_copy(colv, sh_colv.at[pl.ds(tb + chunk * TBL, TBL)])
    pltpu.sync_copy(coli, sh_coli.at[pl.ds(tb + chunk * TBL, TBL)])
    pltpu.sync_copy(scat, sh_scat.at[pl.ds(tb + chunk * TBL, TBL)])
    plsc.subcore_barrier()

    pltpu.sync_copy(sh_colv.at[pl.ds(tb, N_CHUNKS * TBL)], mv)
    pltpu.sync_copy(sh_coli.at[pl.ds(tb, N_CHUNKS * TBL)], mi)
    pltpu.sync_copy(sh_scat.at[pl.ds(tb, N_CHUNKS * TBL)], ms)

    lane0 = lanes == 0
    zi = jnp.zeros((L,), _i32)

    def _merge(g, _):
        off = g * L
        bv = mv[pl.ds(off, L)]
        bi = mi[pl.ds(off, L)]
        sm = ms[pl.ds(off, L)]
        for ch in range(1, N_CHUNKS):
            coff = ch * N_GT * L + off
            cv = mv[pl.ds(coff, L)]
            ci = mi[pl.ds(coff, L)]
            cb = cv > bv
            bv = jnp.where(cb, cv, bv)
            bi = jnp.where(cb, ci, bi)
            sm = jnp.maximum(sm, ms[pl.ds(coff, L)])
        cmax = jnp.max(bv)
        cidx = jnp.min(jnp.where(bv == cmax, bi, BIG_I))
        sg = jnp.max(sm)
        ga = jnp.where(sg >= 0, sg, cidx)
        plsc.store_scatter(gt_arg, [zi + g], zi + ga, mask=lane0)
        return 0

    lax.fori_loop(0, N_GT, _merge, 0)

    ones_i = jnp.ones((L,), _i32)
    for gb in range(N_GT // L):
        ga_v = gt_arg[pl.ds(gb * L, L)]
        inm = (ga_v >= base) & (ga_v < base + CHUNK)
        li = jnp.where(inm, ga_v - base, 0)
        plsc.store_scatter(flags, [li], ones_i, mask=inm)

    def _delta(i, acc):
        a_pos, a_rl, a_val, a_ce = acc
        off = i * L
        aidx = base + off + lanes
        valid = aidx < N_ANCHOR
        fl = flags[pl.ds(off, L)] > 0
        mi_v = miou[pl.ds(off, L)]
        posm = (mi_v >= POS_IOU) | fl
        validm = posm | ((mi_v < NEG_IOU) & valid)
        a_pos = a_pos + jnp.where(posm, 1.0, 0.0)
        a_rl = a_rl + jnp.where(posm, rls[pl.ds(off, L)], 0.0)
        a_val = a_val + jnp.where(validm, 1.0, 0.0)
        ce = jnp.where(posm, ce1s[pl.ds(off, L)], ce0s[pl.ds(off, L)])
        a_ce = a_ce + jnp.where(validm, ce, 0.0)
        return (a_pos, a_rl, a_val, a_ce)

    zero = jnp.zeros((L,), _f32)
    a_pos, a_rl, a_val, a_ce = lax.fori_loop(
        0, N_VEC, _delta, (zero, zero, zero, zero))
    stage[pl.ds(0, L)] = a_pos
    stage[pl.ds(L, L)] = a_rl
    stage[pl.ds(2 * L, L)] = a_val
    stage[pl.ds(3 * L, L)] = a_ce
    pltpu.sync_copy(stage, sh_sums.at[pl.ds((img_l * N_CHUNKS + chunk) * 4 * L,
                                            4 * L)])
    plsc.subcore_barrier()

    @pl.when(s == 0)
    def _finalize():
        total = jnp.zeros((L,), _f32)
        for il in range(2):
            pltpu.sync_copy(
                sh_sums.at[pl.ds(il * N_CHUNKS * 4 * L, N_CHUNKS * 4 * L)], fin)
            t_pos = jnp.zeros((L,), _f32)
            t_rl = jnp.zeros((L,), _f32)
            t_val = jnp.zeros((L,), _f32)
            t_ce = jnp.zeros((L,), _f32)
            for ch in range(N_CHUNKS):
                o = ch * 4 * L
                t_pos = t_pos + fin[pl.ds(o, L)]
                t_rl = t_rl + fin[pl.ds(o + L, L)]
                t_val = t_val + fin[pl.ds(o + 2 * L, L)]
                t_ce = t_ce + fin[pl.ds(o + 3 * L, L)]
            zf = jnp.zeros((L,), _f32)
            num_pos = jnp.maximum(zf + jnp.sum(t_pos), 1.0)
            num_val = jnp.maximum(zf + jnp.sum(t_val), 1.0)
            total = (total + (zf + jnp.sum(t_rl)) / num_pos
                     + (zf + jnp.sum(t_ce)) / num_val)
        outv[...] = total
        pltpu.sync_copy(outv, out.at[pl.ds(c * L, L)])


@jax.jit
def kernel(anchors, bboxes, rpn_loc, rpn_score):
    pad = N_CHUNKS * CHUNK - N_ANCHOR
    anc = jnp.pad(anchors.astype(_f32).T, ((0, 0), (0, pad))).reshape(-1)
    bbox = jnp.transpose(bboxes.astype(_f32), (0, 2, 1)).reshape(-1)
    loc = jnp.pad(jnp.transpose(rpn_loc, (0, 2, 1)),
                  ((0, 0), (0, 0), (0, pad))).reshape(-1)
    score = jnp.pad(jnp.transpose(rpn_score, (0, 2, 1)),
                    ((0, 0), (0, 0), (0, pad))).reshape(-1)

    mesh = plsc.VectorSubcoreMesh(core_axis_name="c", subcore_axis_name="s",
                                  num_cores=2, num_subcores=16)
    run = pl.kernel(
        _sc_body,
        out_type=jax.ShapeDtypeStruct((2 * L,), _f32),
        mesh=mesh,
        compiler_params=pltpu.CompilerParams(needs_layout_passes=False),
        scratch_types=[
            pltpu.VMEM((CHUNK,), _f32), pltpu.VMEM((CHUNK,), _f32),
            pltpu.VMEM((CHUNK,), _f32), pltpu.VMEM((CHUNK,), _f32),
            pltpu.VMEM((CHUNK,), _f32), pltpu.VMEM((CHUNK,), _f32),
            pltpu.VMEM((CHUNK,), _f32), pltpu.VMEM((CHUNK,), _f32),
            pltpu.VMEM((CHUNK,), _f32), pltpu.VMEM((CHUNK,), _f32),
            pltpu.VMEM((N_GT,), _f32), pltpu.VMEM((N_GT,), _f32),
            pltpu.VMEM((N_GT,), _f32), pltpu.VMEM((N_GT,), _f32),
            pltpu.VMEM((CHUNK,), _f32), pltpu.VMEM((CHUNK,), _i32),
            pltpu.VMEM((CHUNK,), _f32), pltpu.VMEM((CHUNK,), _f32),
            pltpu.VMEM((CHUNK,), _f32),
            pltpu.VMEM((CHUNK,), _i32),
            pltpu.VMEM((N_GT * L,), _f32), pltpu.VMEM((N_GT * L,), _i32),
            pltpu.VMEM((N_GT * L,), _i32),
            pltpu.VMEM((N_CHUNKS * N_GT * L,), _f32),
            pltpu.VMEM((N_CHUNKS * N_GT * L,), _i32),
            pltpu.VMEM((N_CHUNKS * N_GT * L,), _i32),
            pltpu.VMEM((N_GT,), _i32),
            pltpu.VMEM((4 * L,), _f32),
            pltpu.VMEM((N_CHUNKS * 4 * L,), _f32),
            pltpu.VMEM((L,), _f32),
            pltpu.VMEM_SHARED((2 * N_CHUNKS * N_GT * L,), _f32),
            pltpu.VMEM_SHARED((2 * N_CHUNKS * N_GT * L,), _i32),
            pltpu.VMEM_SHARED((2 * N_CHUNKS * N_GT * L,), _i32),
            pltpu.VMEM_SHARED((2 * N_CHUNKS * 4 * L,), _f32),
        ],
    )
    out = run(anc, bbox, loc, score)
    return out[0] + out[L]
